# baseline (speedup 1.0000x reference)
import functools

import jax
import jax.numpy as jnp
from jax import lax
from jax.experimental import pallas as pl
from jax.experimental.pallas import tpu as pltpu

P = 32
G = 8
MAX_RADIUS = 2.0
ANG_TO_NM = 0.1

_TN = (((0,), (0,)), ((), ()))


def _egnn_body(L, BLOCK, DE, HID, D, coeff,
               ct_ref, aidx_ref, emb_ref,
               w1a_ref, w1b_ref, w1c_ref, w2_ref, b2_ref,
               na_ref, nb_ref, nb1_ref, nw2_ref, nb2_ref,
               out_ref):
    f32 = jnp.float32
    NG = G * P
    PP = P * P

    e_id = lax.broadcasted_iota(jnp.int32, (1, PP), 1)
    i_id = lax.shift_right_logical(e_id, 5)
    j_id = lax.bitwise_and(e_id, P - 1)
    nr = lax.broadcasted_iota(jnp.int32, (P, PP), 0)
    repd = ((i_id == nr).astype(f32) - (j_id == nr).astype(f32))
    ec = lax.broadcasted_iota(jnp.int32, (2 * P, PP), 1)
    er = lax.broadcasted_iota(jnp.int32, (2 * P, PP), 0)
    ohijT = (((ec // P) == er) | ((ec % P) == (er - P))).astype(f32)

    offc = (MAX_RADIUS / (DE - 1)) * lax.broadcasted_iota(
        jnp.int32, (DE, 1), 0).astype(f32)
    ones_row = jnp.ones((1, PP), f32)
    one11 = jnp.ones((1, 1), f32)

    emcols, lhsTs = [], []
    for g in range(G):
        df = jnp.dot(ct_ref[g], repd, preferred_element_type=f32,
                     precision=lax.Precision.HIGHEST)
        d2 = (df[0:1, :] ** 2 + df[1:2, :] ** 2 + df[2:3, :] ** 2)
        emt = ((d2 < MAX_RADIUS * MAX_RADIUS) & (i_id != j_id)
               & (j_id < BLOCK)).astype(f32)
        dist = jnp.sqrt(d2 + 1e-12)
        ea = jnp.exp(coeff * (dist - offc) ** 2)
        emcols.append(lax.dot_general(emt, one11, _TN,
                                      preferred_element_type=f32))
        lhsTs.append(jnp.concatenate([d2, ea, ones_row, ohijT], axis=0))

    aidx = aidx_ref[...]
    oneh = (aidx == lax.broadcasted_iota(jnp.int32, (1, 128), 1)).astype(f32)
    h = jnp.dot(oneh, emb_ref[...], preferred_element_type=f32)

    for l in range(L):
        hr = jnp.dot(h, w1a_ref[l], preferred_element_type=f32)
        hc = jnp.dot(h, w1b_ref[l], preferred_element_type=f32)
        aggs = []
        for g in range(G):
            rhs = jnp.concatenate(
                [w1c_ref[l], hr[g * P:(g + 1) * P, :],
                 hc[g * P:(g + 1) * P, :]], axis=0)
            m1 = jax.nn.relu(lax.dot_general(lhsTs[g], rhs, _TN,
                                             preferred_element_type=f32))
            m2 = jax.nn.relu(jnp.dot(m1, w2_ref[l], preferred_element_type=f32)
                             + b2_ref[l:l + 1, :])
            m2 = m2 * emcols[g]
            aggs.append(m2.reshape(P, P, HID).sum(axis=1))
        agg = jnp.concatenate(aggs, axis=0)
        nm = jax.nn.relu(jnp.dot(h, na_ref[l], preferred_element_type=f32)
                         + jnp.dot(agg, nb_ref[l], preferred_element_type=f32)
                         + nb1_ref[l:l + 1, :])
        nm = jnp.dot(nm, nw2_ref[l], preferred_element_type=f32) + nb2_ref[l:l + 1, :]
        h = 2.0 * h + nm

    pooled = h.reshape(G, P, D)[:, :BLOCK, :].sum(axis=1) * (1.0 / BLOCK)
    out_ref[...] = pooled.reshape(1, G, D)


def kernel(local_coords, atom_types, bb_num_vec, emb, edge_w1, edge_b1,
           edge_w2, edge_b2, node_w1, node_b1, node_w2, node_b2,
           coord_w1, coord_b1, coord_w2):
    f32 = jnp.float32
    N = local_coords.shape[0]
    NB = bb_num_vec.shape[0]
    BLOCK = N // NB
    D = emb.shape[1]
    HID = edge_w2.shape[1]
    L = edge_w1.shape[0]
    DE = edge_w1.shape[1] - 2 * D - 1
    coeff = -0.5 / (MAX_RADIUS / (DE - 1)) ** 2
    NC = NB // G
    PP = P * P

    lc3 = local_coords.astype(f32).reshape(NB, BLOCK, 3)
    lcp = jnp.pad(lc3, ((0, 0), (0, P - BLOCK), (0, 0)))
    ct = lcp.transpose(0, 2, 1)

    ai = (atom_types.astype(jnp.int32) - 1) % emb.shape[0]
    aip = jnp.pad(ai.reshape(NB, BLOCK), ((0, 0), (0, P - BLOCK)))
    aip = aip.reshape(NB * P, 1)

    embp = jnp.pad(emb.astype(f32), ((0, 128 - emb.shape[0]), (0, 0)))

    w1a = edge_w1[:, :D, :]
    w1b = edge_w1[:, D:2 * D, :]
    w1c = jnp.concatenate([
        (ANG_TO_NM * ANG_TO_NM) * edge_w1[:, 2 * D:2 * D + 1, :],
        edge_w1[:, 2 * D + 1:, :],
        edge_b1[:, None, :],
    ], axis=1)
    na = node_w1[:, :D, :]
    nb = node_w1[:, D:, :]

    body = functools.partial(_egnn_body, L, BLOCK, DE, HID, D, coeff)
    out = pl.pallas_call(
        body,
        grid=(NC,),
        in_specs=[
            pl.BlockSpec((G, 3, P), lambda b: (b, 0, 0)),
            pl.BlockSpec((G * P, 1), lambda b: (b, 0)),
            pl.BlockSpec((128, 128), lambda b: (0, 0)),
            pl.BlockSpec((L, D, HID), lambda b: (0, 0, 0)),
            pl.BlockSpec((L, D, HID), lambda b: (0, 0, 0)),
            pl.BlockSpec((L, DE + 2, HID), lambda b: (0, 0, 0)),
            pl.BlockSpec((L, HID, HID), lambda b: (0, 0, 0)),
            pl.BlockSpec((L, HID), lambda b: (0, 0)),
            pl.BlockSpec((L, D, HID), lambda b: (0, 0, 0)),
            pl.BlockSpec((L, HID, HID), lambda b: (0, 0, 0)),
            pl.BlockSpec((L, HID), lambda b: (0, 0)),
            pl.BlockSpec((L, HID, D), lambda b: (0, 0, 0)),
            pl.BlockSpec((L, D), lambda b: (0, 0)),
        ],
        out_specs=pl.BlockSpec((1, G, D), lambda b: (b, 0, 0)),
        out_shape=jax.ShapeDtypeStruct((NC, G, D), f32),
        compiler_params=pltpu.CompilerParams(
            dimension_semantics=("arbitrary",)),
    )(ct, aip, embp, w1a, w1b, w1c, edge_w2, edge_b2,
      na, nb, node_b1, node_w2, node_b2)
    return out.reshape(NB, D)


# speedup vs baseline: 1.1210x; 1.1210x over previous
import functools

import jax
import jax.numpy as jnp
from jax import lax
from jax.experimental import pallas as pl
from jax.experimental.pallas import tpu as pltpu

P = 32
G = 16
MAX_RADIUS = 2.0
ANG_TO_NM = 0.1

_TN = (((0,), (0,)), ((), ()))


def _egnn_body(L, BLOCK, DE, HID, D, coeff,
               ct_ref, aidx_ref, emb_ref,
               w1a_ref, w1b_ref, w1c_ref, w2_ref, b2_ref,
               na_ref, nb_ref, nb1_ref, nw2_ref, nb2_ref,
               out_ref):
    f32 = jnp.float32
    NG = G * P
    PP = P * P

    e_id = lax.broadcasted_iota(jnp.int32, (1, PP), 1)
    i_id = lax.shift_right_logical(e_id, 5)
    j_id = lax.bitwise_and(e_id, P - 1)
    nr = lax.broadcasted_iota(jnp.int32, (P, PP), 0)
    repd = ((i_id == nr).astype(f32) - (j_id == nr).astype(f32))
    ec = lax.broadcasted_iota(jnp.int32, (2 * P, PP), 1)
    er = lax.broadcasted_iota(jnp.int32, (2 * P, PP), 0)
    ohijT = (((ec // P) == er) | ((ec % P) == (er - P))).astype(f32)

    offc = (MAX_RADIUS / (DE - 1)) * lax.broadcasted_iota(
        jnp.int32, (DE, 1), 0).astype(f32)
    ones_row = jnp.ones((1, PP), f32)
    one11 = jnp.ones((1, 1), f32)

    emcols, lhsTs = [], []
    for g in range(G):
        df = jnp.dot(ct_ref[g], repd, preferred_element_type=f32,
                     precision=lax.Precision.HIGHEST)
        d2 = (df[0:1, :] ** 2 + df[1:2, :] ** 2 + df[2:3, :] ** 2)
        emt = ((d2 < MAX_RADIUS * MAX_RADIUS) & (i_id != j_id)
               & (j_id < BLOCK)).astype(f32)
        dist = jnp.sqrt(d2 + 1e-12)
        ea = jnp.exp(coeff * (dist - offc) ** 2)
        emcols.append(lax.dot_general(emt, one11, _TN,
                                      preferred_element_type=f32))
        lhsTs.append(jnp.concatenate([d2, ea, ones_row, ohijT], axis=0))

    aidx = aidx_ref[...]
    oneh = (aidx == lax.broadcasted_iota(jnp.int32, (1, 128), 1)).astype(f32)
    h = jnp.dot(oneh, emb_ref[...], preferred_element_type=f32)

    for l in range(L):
        hr = jnp.dot(h, w1a_ref[l], preferred_element_type=f32)
        hc = jnp.dot(h, w1b_ref[l], preferred_element_type=f32)
        aggs = []
        for g in range(G):
            rhs = jnp.concatenate(
                [w1c_ref[l], hr[g * P:(g + 1) * P, :],
                 hc[g * P:(g + 1) * P, :]], axis=0)
            m1 = jax.nn.relu(lax.dot_general(lhsTs[g], rhs, _TN,
                                             preferred_element_type=f32))
            m2 = jax.nn.relu(jnp.dot(m1, w2_ref[l], preferred_element_type=f32)
                             + b2_ref[l:l + 1, :])
            m2 = m2 * emcols[g]
            aggs.append(m2.reshape(P, P, HID).sum(axis=1))
        agg = jnp.concatenate(aggs, axis=0)
        nm = jax.nn.relu(jnp.dot(h, na_ref[l], preferred_element_type=f32)
                         + jnp.dot(agg, nb_ref[l], preferred_element_type=f32)
                         + nb1_ref[l:l + 1, :])
        nm = jnp.dot(nm, nw2_ref[l], preferred_element_type=f32) + nb2_ref[l:l + 1, :]
        h = 2.0 * h + nm

    pooled = h.reshape(G, P, D)[:, :BLOCK, :].sum(axis=1) * (1.0 / BLOCK)
    out_ref[...] = pooled.reshape(1, G, D)


def kernel(local_coords, atom_types, bb_num_vec, emb, edge_w1, edge_b1,
           edge_w2, edge_b2, node_w1, node_b1, node_w2, node_b2,
           coord_w1, coord_b1, coord_w2):
    f32 = jnp.float32
    N = local_coords.shape[0]
    NB = bb_num_vec.shape[0]
    BLOCK = N // NB
    D = emb.shape[1]
    HID = edge_w2.shape[1]
    L = edge_w1.shape[0]
    DE = edge_w1.shape[1] - 2 * D - 1
    coeff = -0.5 / (MAX_RADIUS / (DE - 1)) ** 2
    NC = NB // G
    PP = P * P

    lc3 = local_coords.astype(f32).reshape(NB, BLOCK, 3)
    lcp = jnp.pad(lc3, ((0, 0), (0, P - BLOCK), (0, 0)))
    ct = lcp.transpose(0, 2, 1)

    ai = (atom_types.astype(jnp.int32) - 1) % emb.shape[0]
    aip = jnp.pad(ai.reshape(NB, BLOCK), ((0, 0), (0, P - BLOCK)))
    aip = aip.reshape(NB * P, 1)

    embp = jnp.pad(emb.astype(f32), ((0, 128 - emb.shape[0]), (0, 0)))

    w1a = edge_w1[:, :D, :]
    w1b = edge_w1[:, D:2 * D, :]
    w1c = jnp.concatenate([
        (ANG_TO_NM * ANG_TO_NM) * edge_w1[:, 2 * D:2 * D + 1, :],
        edge_w1[:, 2 * D + 1:, :],
        edge_b1[:, None, :],
    ], axis=1)
    na = node_w1[:, :D, :]
    nb = node_w1[:, D:, :]

    body = functools.partial(_egnn_body, L, BLOCK, DE, HID, D, coeff)
    out = pl.pallas_call(
        body,
        grid=(NC,),
        in_specs=[
            pl.BlockSpec((G, 3, P), lambda b: (b, 0, 0)),
            pl.BlockSpec((G * P, 1), lambda b: (b, 0)),
            pl.BlockSpec((128, 128), lambda b: (0, 0)),
            pl.BlockSpec((L, D, HID), lambda b: (0, 0, 0)),
            pl.BlockSpec((L, D, HID), lambda b: (0, 0, 0)),
            pl.BlockSpec((L, DE + 2, HID), lambda b: (0, 0, 0)),
            pl.BlockSpec((L, HID, HID), lambda b: (0, 0, 0)),
            pl.BlockSpec((L, HID), lambda b: (0, 0)),
            pl.BlockSpec((L, D, HID), lambda b: (0, 0, 0)),
            pl.BlockSpec((L, HID, HID), lambda b: (0, 0, 0)),
            pl.BlockSpec((L, HID), lambda b: (0, 0)),
            pl.BlockSpec((L, HID, D), lambda b: (0, 0, 0)),
            pl.BlockSpec((L, D), lambda b: (0, 0)),
        ],
        out_specs=pl.BlockSpec((1, G, D), lambda b: (b, 0, 0)),
        out_shape=jax.ShapeDtypeStruct((NC, G, D), f32),
        compiler_params=pltpu.CompilerParams(
            dimension_semantics=("arbitrary",)),
    )(ct, aip, embp, w1a, w1b, w1c, edge_w2, edge_b2,
      na, nb, node_b1, node_w2, node_b2)
    return out.reshape(NB, D)
